# Initial kernel scaffold; baseline (speedup 1.0000x reference)
import jax
import jax.numpy as jnp
from jax.experimental import pallas as pl


def kernel(x, edge_index, edge_attr, batch, We0, be0, W0, b0, We1, be1, W1, b1, We2, be2, W2, b2, L0W, L0b, L1W, L1b):
    raise NotImplementedError("write your pallas kernel here")


# speedup vs baseline: 1.8450x; 1.8450x over previous
import functools

import jax
import jax.numpy as jnp
from jax import lax
from jax.experimental import pallas as pl
from jax.experimental.pallas import tpu as pltpu
from jax.experimental.pallas import tpu_sc as plsc

N = 10000
NP = 10240
E = 320000
D = 128
DE = 16
H = 128
C = 32
G = 128

NC = 2
NS = 16
NW = NC * NS
E_PER_W = E // NW
CHUNK = 80
NCHUNK = E_PER_W // CHUNK
ROWS_PER_SUB = NP // NS


IGRP = 25
NGRP = NCHUNK // IGRP


def _sc_layer_body(h_hbm, src_hbm, dst_hbm, ep_hbm, zero_hbm, out_hbm,
                   src_v, dst_v, rows_v, ep_v, acc_sh, sem):
    cid = lax.axis_index("c")
    sid = lax.axis_index("s")
    wid = sid * NC + cid

    pltpu.sync_copy(zero_hbm.at[pl.ds(sid * ROWS_PER_SUB, ROWS_PER_SUB)],
                    acc_sh.at[pl.ds(sid * ROWS_PER_SUB, ROWS_PER_SUB)])
    plsc.subcore_barrier()

    ep_base = wid * E_PER_W

    def grp_body(g, carry0):
        pltpu.sync_copy(src_hbm.at[wid, g], src_v)
        pltpu.sync_copy(dst_hbm.at[wid, g], dst_v)

        def chunk_body(c, carry):
            pltpu.async_copy(h_hbm.at[src_v.at[c]], rows_v, sem).wait()
            pltpu.sync_copy(
                ep_hbm.at[pl.ds(ep_base + (g * IGRP + c) * CHUNK, CHUNK)],
                ep_v)

            def row_body(r, carry2):
                for j in range(D // 16):
                    s = pl.ds(j * 16, 16)
                    v = rows_v[r, s] + ep_v[r, s]
                    rows_v[r, s] = jnp.maximum(v, 0.0)
                return carry2

            lax.fori_loop(0, CHUNK, row_body, 0, unroll=4)
            pltpu.sync_copy(rows_v, acc_sh.at[dst_v.at[c]], add=True)
            return carry

        lax.fori_loop(0, IGRP, chunk_body, 0)
        return carry0

    lax.fori_loop(0, NGRP, grp_body, 0)
    plsc.subcore_barrier()
    pltpu.sync_copy(acc_sh.at[pl.ds(sid * ROWS_PER_SUB, ROWS_PER_SUB)],
                    out_hbm.at[cid, pl.ds(sid * ROWS_PER_SUB, ROWS_PER_SUB)])


_sc_layer = pl.kernel(
    _sc_layer_body,
    out_type=jax.ShapeDtypeStruct((NC, NP, D), jnp.float32),
    mesh=plsc.VectorSubcoreMesh(core_axis_name="c", subcore_axis_name="s"),
    scratch_types=[
        pltpu.VMEM((IGRP, CHUNK), jnp.int32),
        pltpu.VMEM((IGRP, CHUNK), jnp.int32),
        pltpu.VMEM((CHUNK, D), jnp.float32),
        pltpu.VMEM((CHUNK, D), jnp.float32),
        pltpu.VMEM_SHARED((NP, D), jnp.float32),
        pltpu.SemaphoreType.DMA,
    ],
)


def _eproj_body(ea_ref, w_ref, b_ref, o0_ref, o1_ref, o2_ref):
    prod = jnp.dot(ea_ref[...], w_ref[...],
                   preferred_element_type=jnp.float32) + b_ref[...]
    o0_ref[...] = prod[:, 0:D]
    o1_ref[...] = prod[:, D:2 * D]
    o2_ref[...] = prod[:, 2 * D:3 * D]


_EBLK = 2000


def _eproj(edge_attr, wcat, bcat):
    return pl.pallas_call(
        _eproj_body,
        grid=(E // _EBLK,),
        in_specs=[
            pl.BlockSpec((_EBLK, DE), lambda i: (i, 0)),
            pl.BlockSpec((DE, 3 * D), lambda i: (0, 0)),
            pl.BlockSpec((1, 3 * D), lambda i: (0, 0)),
        ],
        out_specs=[
            pl.BlockSpec((_EBLK, D), lambda i: (i, 0)),
            pl.BlockSpec((_EBLK, D), lambda i: (i, 0)),
            pl.BlockSpec((_EBLK, D), lambda i: (i, 0)),
        ],
        out_shape=[jax.ShapeDtypeStruct((E, D), jnp.float32)] * 3,
    )(edge_attr, wcat, bcat)


def _update_body(h_ref, agg_ref, w_ref, b_ref, o_ref):
    s = h_ref[...] + agg_ref[0] + agg_ref[1]
    m = jnp.dot(s, w_ref[...], preferred_element_type=jnp.float32) + b_ref[...]
    o_ref[...] = jnp.where(m > 0, m, 0.01 * m)


_NBLK = 1024


def _update(h, aggs, w, b):
    return pl.pallas_call(
        _update_body,
        grid=(NP // _NBLK,),
        in_specs=[
            pl.BlockSpec((_NBLK, D), lambda i: (i, 0)),
            pl.BlockSpec((NC, _NBLK, D), lambda i: (0, i, 0)),
            pl.BlockSpec((D, H), lambda i: (0, 0)),
            pl.BlockSpec((1, H), lambda i: (0, 0)),
        ],
        out_specs=pl.BlockSpec((_NBLK, H), lambda i: (i, 0)),
        out_shape=jax.ShapeDtypeStruct((NP, H), jnp.float32),
    )(h, aggs, w, b)


def _pool_head_body(h_ref, batch_ref, l0w_ref, l0b_ref, l1w_ref, l1b_ref,
                    o_ref, sums_ref, cnts_ref):
    i = pl.program_id(0)

    @pl.when(i == 0)
    def _():
        sums_ref[...] = jnp.zeros_like(sums_ref)
        cnts_ref[...] = jnp.zeros_like(cnts_ref)

    gids = lax.broadcasted_iota(jnp.int32, (G, G), 0)
    mask = (gids == batch_ref[0]).astype(jnp.float32)
    sums_ref[...] += jnp.dot(mask, h_ref[...],
                             preferred_element_type=jnp.float32)
    cnts_ref[...] += jnp.dot(mask, jnp.ones((G, H), jnp.float32),
                             preferred_element_type=jnp.float32)

    @pl.when(i == pl.num_programs(0) - 1)
    def _():
        pooled = sums_ref[...] / jnp.maximum(cnts_ref[...], 1.0)
        z = jnp.dot(pooled, l0w_ref[...],
                    preferred_element_type=jnp.float32) + l0b_ref[...]
        z = jnp.where(z > 0, z, 0.01 * z)
        z = jnp.dot(z, l1w_ref[...],
                    preferred_element_type=jnp.float32) + l1b_ref[...]
        m = jnp.max(z, axis=1, keepdims=True)
        zs = z - m
        lse = jnp.log(jnp.sum(jnp.exp(zs), axis=1, keepdims=True))
        o_ref[...] = zs - lse


def _pool_head(h, batch2d, l0w, l0b, l1w, l1b):
    return pl.pallas_call(
        _pool_head_body,
        grid=(NP // G,),
        in_specs=[
            pl.BlockSpec((G, H), lambda i: (i, 0)),
            pl.BlockSpec((1, 1, G), lambda i: (i, 0, 0)),
            pl.BlockSpec((H, H), lambda i: (0, 0)),
            pl.BlockSpec((1, H), lambda i: (0, 0)),
            pl.BlockSpec((H, C), lambda i: (0, 0)),
            pl.BlockSpec((1, C), lambda i: (0, 0)),
        ],
        out_specs=pl.BlockSpec((G, C), lambda i: (0, 0)),
        out_shape=jax.ShapeDtypeStruct((G, C), jnp.float32),
        scratch_shapes=[
            pltpu.VMEM((G, H), jnp.float32),
            pltpu.VMEM((G, H), jnp.float32),
        ],
    )(h, batch2d, l0w, l0b, l1w, l1b)


def kernel(x, edge_index, edge_attr, batch,
           We0, be0, W0, b0,
           We1, be1, W1, b1,
           We2, be2, W2, b2,
           L0W, L0b, L1W, L1b):
    src = edge_index[0].astype(jnp.int32).reshape(NW, NGRP, IGRP, CHUNK)
    dst = edge_index[1].astype(jnp.int32).reshape(NW, NGRP, IGRP, CHUNK)

    xp = jnp.zeros((NP, D), jnp.float32).at[:N].set(x)
    zeros_np = jnp.zeros((NP, D), jnp.float32)

    wcat = jnp.concatenate([We0, We1, We2], axis=1)
    bcat = jnp.concatenate([be0, be1, be2]).reshape(1, 3 * D)
    eps = _eproj(edge_attr, wcat, bcat)

    h = xp
    for ep, w, b in zip(eps, (W0, W1, W2), (b0, b1, b2)):
        aggs = _sc_layer(h, src, dst, ep, zeros_np)
        h = _update(h, aggs, w, b.reshape(1, H))

    batchp = jnp.full((NP,), G, jnp.int32).at[:N].set(batch.astype(jnp.int32))
    out = _pool_head(h, batchp.reshape(NP // G, 1, G),
                     L0W, L0b.reshape(1, H), L1W, L1b.reshape(1, C))
    return out


# speedup vs baseline: 2.4476x; 1.3266x over previous
import functools

import jax
import jax.numpy as jnp
from jax import lax
from jax.experimental import pallas as pl
from jax.experimental.pallas import tpu as pltpu
from jax.experimental.pallas import tpu_sc as plsc

N = 10000
NP = 10240
E = 320000
D = 128
DE = 16
H = 128
C = 32
G = 128

NC = 2
NS = 16
NW = NC * NS
E_PAD = 327680
E_PER_W = E_PAD // NW
CHUNK = 40
NCHUNK = E_PER_W // CHUNK
NGRP = 4
GCH = NCHUNK // NGRP
NPAIR = GCH // 2
ROWS_PER_SUB = NP // NS


def _sc_layer_body(h_hbm, src_hbm, dst_hbm, ep_hbm, zero_hbm, out_hbm,
                   src_v, dst_v, rows0, rows1, ep0, ep1, acc_sh,
                   g0, g1, e0, e1):
    cid = lax.axis_index("c")
    sid = lax.axis_index("s")
    wid = sid * NC + cid

    pltpu.sync_copy(zero_hbm.at[pl.ds(sid * ROWS_PER_SUB, ROWS_PER_SUB)],
                    acc_sh.at[pl.ds(sid * ROWS_PER_SUB, ROWS_PER_SUB)])
    plsc.subcore_barrier()

    ep_base = wid * E_PER_W

    def compute(rows_v, ep_v):
        def row_body(r, carry2):
            for j in range(D // 16):
                s = pl.ds(j * 16, 16)
                v = rows_v[r, s] + ep_v[r, s]
                rows_v[r, s] = jnp.maximum(v, 0.0)
            return carry2

        lax.fori_loop(0, CHUNK, row_body, 0, unroll=4)

    for grp in range(NGRP):
        goff = grp * GCH

        def issue(c_grp, rows_v, ep_v, gsem, esem, goff=goff):
            pltpu.async_copy(h_hbm.at[src_v.at[c_grp]], rows_v, gsem)
            pltpu.async_copy(
                ep_hbm.at[pl.ds(ep_base + (goff + c_grp) * CHUNK, CHUNK)],
                ep_v, esem)

        pltpu.sync_copy(src_hbm.at[wid, grp], src_v)
        pltpu.sync_copy(dst_hbm.at[wid, grp], dst_v)
        issue(0, rows0, ep0, g0, e0)

        def pair_body(k, carry):
            c_even = 2 * k
            c_odd = 2 * k + 1
            issue(c_odd, rows1, ep1, g1, e1)
            pltpu.make_async_copy(h_hbm.at[src_v.at[c_even]], rows0, g0).wait()
            pltpu.make_async_copy(
                ep_hbm.at[pl.ds(ep_base, CHUNK)], ep0, e0).wait()
            compute(rows0, ep0)
            pltpu.sync_copy(rows0, acc_sh.at[dst_v.at[c_even]], add=True)

            @pl.when(k < NPAIR - 1)
            def _():
                issue(c_even + 2, rows0, ep0, g0, e0)

            pltpu.make_async_copy(h_hbm.at[src_v.at[c_odd]], rows1, g1).wait()
            pltpu.make_async_copy(
                ep_hbm.at[pl.ds(ep_base, CHUNK)], ep1, e1).wait()
            compute(rows1, ep1)
            pltpu.sync_copy(rows1, acc_sh.at[dst_v.at[c_odd]], add=True)
            return carry

        lax.fori_loop(0, NPAIR, pair_body, 0)

    plsc.subcore_barrier()
    pltpu.sync_copy(acc_sh.at[pl.ds(sid * ROWS_PER_SUB, ROWS_PER_SUB)],
                    out_hbm.at[cid, pl.ds(sid * ROWS_PER_SUB, ROWS_PER_SUB)])


_sc_layer = pl.kernel(
    _sc_layer_body,
    out_type=jax.ShapeDtypeStruct((NC, NP, D), jnp.float32),
    mesh=plsc.VectorSubcoreMesh(core_axis_name="c", subcore_axis_name="s"),
    scratch_types=[
        pltpu.VMEM((GCH, CHUNK), jnp.int32),
        pltpu.VMEM((GCH, CHUNK), jnp.int32),
        pltpu.VMEM((CHUNK, D), jnp.float32),
        pltpu.VMEM((CHUNK, D), jnp.float32),
        pltpu.VMEM((CHUNK, D), jnp.float32),
        pltpu.VMEM((CHUNK, D), jnp.float32),
        pltpu.VMEM_SHARED((NP, D), jnp.float32),
        pltpu.SemaphoreType.DMA,
        pltpu.SemaphoreType.DMA,
        pltpu.SemaphoreType.DMA,
        pltpu.SemaphoreType.DMA,
    ],
)


def _eproj_body(ea_ref, w_ref, b_ref, o0_ref, o1_ref, o2_ref):
    prod = jnp.dot(ea_ref[...], w_ref[...],
                   preferred_element_type=jnp.float32) + b_ref[...]
    o0_ref[...] = prod[:, 0:D]
    o1_ref[...] = prod[:, D:2 * D]
    o2_ref[...] = prod[:, 2 * D:3 * D]


_EBLK = 2048


def _eproj(edge_attr, wcat, bcat):
    return pl.pallas_call(
        _eproj_body,
        grid=(E_PAD // _EBLK,),
        in_specs=[
            pl.BlockSpec((_EBLK, DE), lambda i: (i, 0)),
            pl.BlockSpec((DE, 3 * D), lambda i: (0, 0)),
            pl.BlockSpec((1, 3 * D), lambda i: (0, 0)),
        ],
        out_specs=[
            pl.BlockSpec((_EBLK, D), lambda i: (i, 0)),
            pl.BlockSpec((_EBLK, D), lambda i: (i, 0)),
            pl.BlockSpec((_EBLK, D), lambda i: (i, 0)),
        ],
        out_shape=[jax.ShapeDtypeStruct((E_PAD, D), jnp.float32)] * 3,
    )(edge_attr, wcat, bcat)


def _update_body(h_ref, agg_ref, w_ref, b_ref, o_ref):
    s = h_ref[...] + agg_ref[0] + agg_ref[1]
    m = jnp.dot(s, w_ref[...], preferred_element_type=jnp.float32) + b_ref[...]
    o_ref[...] = jnp.where(m > 0, m, 0.01 * m)


_NBLK = 1024


def _update(h, aggs, w, b):
    return pl.pallas_call(
        _update_body,
        grid=(NP // _NBLK,),
        in_specs=[
            pl.BlockSpec((_NBLK, D), lambda i: (i, 0)),
            pl.BlockSpec((NC, _NBLK, D), lambda i: (0, i, 0)),
            pl.BlockSpec((D, H), lambda i: (0, 0)),
            pl.BlockSpec((1, H), lambda i: (0, 0)),
        ],
        out_specs=pl.BlockSpec((_NBLK, H), lambda i: (i, 0)),
        out_shape=jax.ShapeDtypeStruct((NP, H), jnp.float32),
    )(h, aggs, w, b)


def _pool_head_body(h_ref, batch_ref, l0w_ref, l0b_ref, l1w_ref, l1b_ref,
                    o_ref, sums_ref, cnts_ref):
    i = pl.program_id(0)

    @pl.when(i == 0)
    def _():
        sums_ref[...] = jnp.zeros_like(sums_ref)
        cnts_ref[...] = jnp.zeros_like(cnts_ref)

    gids = lax.broadcasted_iota(jnp.int32, (G, G), 0)
    mask = (gids == batch_ref[0]).astype(jnp.float32)
    sums_ref[...] += jnp.dot(mask, h_ref[...],
                             preferred_element_type=jnp.float32)
    cnts_ref[...] += jnp.dot(mask, jnp.ones((G, H), jnp.float32),
                             preferred_element_type=jnp.float32)

    @pl.when(i == pl.num_programs(0) - 1)
    def _():
        pooled = sums_ref[...] / jnp.maximum(cnts_ref[...], 1.0)
        z = jnp.dot(pooled, l0w_ref[...],
                    preferred_element_type=jnp.float32) + l0b_ref[...]
        z = jnp.where(z > 0, z, 0.01 * z)
        z = jnp.dot(z, l1w_ref[...],
                    preferred_element_type=jnp.float32) + l1b_ref[...]
        m = jnp.max(z, axis=1, keepdims=True)
        zs = z - m
        lse = jnp.log(jnp.sum(jnp.exp(zs), axis=1, keepdims=True))
        o_ref[...] = zs - lse


def _pool_head(h, batch2d, l0w, l0b, l1w, l1b):
    return pl.pallas_call(
        _pool_head_body,
        grid=(NP // G,),
        in_specs=[
            pl.BlockSpec((G, H), lambda i: (i, 0)),
            pl.BlockSpec((1, 1, G), lambda i: (i, 0, 0)),
            pl.BlockSpec((H, H), lambda i: (0, 0)),
            pl.BlockSpec((1, H), lambda i: (0, 0)),
            pl.BlockSpec((H, C), lambda i: (0, 0)),
            pl.BlockSpec((1, C), lambda i: (0, 0)),
        ],
        out_specs=pl.BlockSpec((G, C), lambda i: (0, 0)),
        out_shape=jax.ShapeDtypeStruct((G, C), jnp.float32),
        scratch_shapes=[
            pltpu.VMEM((G, H), jnp.float32),
            pltpu.VMEM((G, H), jnp.float32),
        ],
    )(h, batch2d, l0w, l0b, l1w, l1b)


def kernel(x, edge_index, edge_attr, batch,
           We0, be0, W0, b0,
           We1, be1, W1, b1,
           We2, be2, W2, b2,
           L0W, L0b, L1W, L1b):
    npad = E_PAD - E
    src = jnp.concatenate(
        [edge_index[0].astype(jnp.int32),
         jnp.arange(npad, dtype=jnp.int32) % N])
    dst = jnp.concatenate(
        [edge_index[1].astype(jnp.int32),
         N + jnp.arange(npad, dtype=jnp.int32) % (NP - N)])
    src = src.reshape(NW, NGRP, GCH, CHUNK)
    dst = dst.reshape(NW, NGRP, GCH, CHUNK)
    eap = jnp.concatenate(
        [edge_attr, jnp.zeros((npad, DE), jnp.float32)])

    xp = jnp.zeros((NP, D), jnp.float32).at[:N].set(x)
    zeros_np = jnp.zeros((NP, D), jnp.float32)

    wcat = jnp.concatenate([We0, We1, We2], axis=1)
    bcat = jnp.concatenate([be0, be1, be2]).reshape(1, 3 * D)
    eps = _eproj(eap, wcat, bcat)

    h = xp
    for ep, w, b in zip(eps, (W0, W1, W2), (b0, b1, b2)):
        aggs = _sc_layer(h, src, dst, ep, zeros_np)
        h = _update(h, aggs, w, b.reshape(1, H))

    batchp = jnp.full((NP,), G, jnp.int32).at[:N].set(batch.astype(jnp.int32))
    out = _pool_head(h, batchp.reshape(NP // G, 1, G),
                     L0W, L0b.reshape(1, H), L1W, L1b.reshape(1, C))
    return out


# speedup vs baseline: 4.1702x; 1.7038x over previous
import functools

import jax
import jax.numpy as jnp
from jax import lax
from jax.experimental import pallas as pl
from jax.experimental.pallas import tpu as pltpu
from jax.experimental.pallas import tpu_sc as plsc

N = 10000
NP = 10240
E = 320000
D = 128
DE = 16
H = 128
C = 32
G = 128

NC = 2
NS = 16
NW = NC * NS
E_PAD = 327680
E_PER_W = E_PAD // NW
CHUNK = 40
NCHUNK = E_PER_W // CHUNK
NGRP = 4
GCH = NCHUNK // NGRP
NPAIR = GCH // 2
ROWS_PER_SUB = NP // NS


def _sc_layer_body(h_hbm, src_hbm, dst_hbm, ep_hbm, zero_hbm, out_hbm,
                   src_v, dst_v, rows0, rows1, ep0, ep1, acc_sh,
                   g0, g1, e0, e1):
    cid = lax.axis_index("c")
    sid = lax.axis_index("s")
    wid = sid * NC + cid

    pltpu.sync_copy(zero_hbm.at[pl.ds(sid * ROWS_PER_SUB, ROWS_PER_SUB)],
                    acc_sh.at[pl.ds(sid * ROWS_PER_SUB, ROWS_PER_SUB)])
    plsc.subcore_barrier()

    ep_base = wid * E_PER_W

    def compute(rows_v, ep_v):
        @plsc.parallel_loop(0, CHUNK, unroll=4)
        def row_body(r):
            for j in range(D // 16):
                s = pl.ds(j * 16, 16)
                v = rows_v[r, s] + ep_v[r, s]
                rows_v[r, s] = jnp.maximum(v, 0.0)

    for grp in range(NGRP):
        goff = grp * GCH

        def issue(c_grp, rows_v, ep_v, gsem, esem, goff=goff):
            pltpu.async_copy(h_hbm.at[src_v.at[c_grp]], rows_v, gsem)
            pltpu.async_copy(
                ep_hbm.at[pl.ds(ep_base + (goff + c_grp) * CHUNK, CHUNK)],
                ep_v, esem)

        pltpu.sync_copy(src_hbm.at[wid, grp], src_v)
        pltpu.sync_copy(dst_hbm.at[wid, grp], dst_v)
        issue(0, rows0, ep0, g0, e0)

        def pair_body(k, carry):
            c_even = 2 * k
            c_odd = 2 * k + 1
            issue(c_odd, rows1, ep1, g1, e1)
            pltpu.make_async_copy(h_hbm.at[src_v.at[c_even]], rows0, g0).wait()
            pltpu.make_async_copy(
                ep_hbm.at[pl.ds(ep_base, CHUNK)], ep0, e0).wait()
            compute(rows0, ep0)
            pltpu.sync_copy(rows0, acc_sh.at[dst_v.at[c_even]], add=True)

            @pl.when(k < NPAIR - 1)
            def _():
                issue(c_even + 2, rows0, ep0, g0, e0)

            pltpu.make_async_copy(h_hbm.at[src_v.at[c_odd]], rows1, g1).wait()
            pltpu.make_async_copy(
                ep_hbm.at[pl.ds(ep_base, CHUNK)], ep1, e1).wait()
            compute(rows1, ep1)
            pltpu.sync_copy(rows1, acc_sh.at[dst_v.at[c_odd]], add=True)
            return carry

        lax.fori_loop(0, NPAIR, pair_body, 0)

    plsc.subcore_barrier()
    pltpu.sync_copy(acc_sh.at[pl.ds(sid * ROWS_PER_SUB, ROWS_PER_SUB)],
                    out_hbm.at[cid, pl.ds(sid * ROWS_PER_SUB, ROWS_PER_SUB)])


_sc_layer = pl.kernel(
    _sc_layer_body,
    out_type=jax.ShapeDtypeStruct((NC, NP, D), jnp.float32),
    mesh=plsc.VectorSubcoreMesh(core_axis_name="c", subcore_axis_name="s"),
    scratch_types=[
        pltpu.VMEM((GCH, CHUNK), jnp.int32),
        pltpu.VMEM((GCH, CHUNK), jnp.int32),
        pltpu.VMEM((CHUNK, D), jnp.float32),
        pltpu.VMEM((CHUNK, D), jnp.float32),
        pltpu.VMEM((CHUNK, D), jnp.float32),
        pltpu.VMEM((CHUNK, D), jnp.float32),
        pltpu.VMEM_SHARED((NP, D), jnp.float32),
        pltpu.SemaphoreType.DMA,
        pltpu.SemaphoreType.DMA,
        pltpu.SemaphoreType.DMA,
        pltpu.SemaphoreType.DMA,
    ],
)


def _eproj_body(ea_ref, w_ref, b_ref, o0_ref, o1_ref, o2_ref):
    prod = jnp.dot(ea_ref[...], w_ref[...],
                   preferred_element_type=jnp.float32) + b_ref[...]
    o0_ref[...] = prod[:, 0:D]
    o1_ref[...] = prod[:, D:2 * D]
    o2_ref[...] = prod[:, 2 * D:3 * D]


_EBLK = 2048


def _eproj(edge_attr, wcat, bcat):
    return pl.pallas_call(
        _eproj_body,
        grid=(E_PAD // _EBLK,),
        in_specs=[
            pl.BlockSpec((_EBLK, DE), lambda i: (i, 0)),
            pl.BlockSpec((DE, 3 * D), lambda i: (0, 0)),
            pl.BlockSpec((1, 3 * D), lambda i: (0, 0)),
        ],
        out_specs=[
            pl.BlockSpec((_EBLK, D), lambda i: (i, 0)),
            pl.BlockSpec((_EBLK, D), lambda i: (i, 0)),
            pl.BlockSpec((_EBLK, D), lambda i: (i, 0)),
        ],
        out_shape=[jax.ShapeDtypeStruct((E_PAD, D), jnp.float32)] * 3,
    )(edge_attr, wcat, bcat)


def _update_body(h_ref, agg_ref, w_ref, b_ref, o_ref):
    s = h_ref[...] + agg_ref[0] + agg_ref[1]
    m = jnp.dot(s, w_ref[...], preferred_element_type=jnp.float32) + b_ref[...]
    o_ref[...] = jnp.where(m > 0, m, 0.01 * m)


_NBLK = 1024


def _update(h, aggs, w, b):
    return pl.pallas_call(
        _update_body,
        grid=(NP // _NBLK,),
        in_specs=[
            pl.BlockSpec((_NBLK, D), lambda i: (i, 0)),
            pl.BlockSpec((NC, _NBLK, D), lambda i: (0, i, 0)),
            pl.BlockSpec((D, H), lambda i: (0, 0)),
            pl.BlockSpec((1, H), lambda i: (0, 0)),
        ],
        out_specs=pl.BlockSpec((_NBLK, H), lambda i: (i, 0)),
        out_shape=jax.ShapeDtypeStruct((NP, H), jnp.float32),
    )(h, aggs, w, b)


def _pool_head_body(h_ref, batch_ref, l0w_ref, l0b_ref, l1w_ref, l1b_ref,
                    o_ref, sums_ref, cnts_ref):
    i = pl.program_id(0)

    @pl.when(i == 0)
    def _():
        sums_ref[...] = jnp.zeros_like(sums_ref)
        cnts_ref[...] = jnp.zeros_like(cnts_ref)

    gids = lax.broadcasted_iota(jnp.int32, (G, G), 0)
    mask = (gids == batch_ref[0]).astype(jnp.float32)
    sums_ref[...] += jnp.dot(mask, h_ref[...],
                             preferred_element_type=jnp.float32)
    cnts_ref[...] += jnp.dot(mask, jnp.ones((G, H), jnp.float32),
                             preferred_element_type=jnp.float32)

    @pl.when(i == pl.num_programs(0) - 1)
    def _():
        pooled = sums_ref[...] / jnp.maximum(cnts_ref[...], 1.0)
        z = jnp.dot(pooled, l0w_ref[...],
                    preferred_element_type=jnp.float32) + l0b_ref[...]
        z = jnp.where(z > 0, z, 0.01 * z)
        z = jnp.dot(z, l1w_ref[...],
                    preferred_element_type=jnp.float32) + l1b_ref[...]
        m = jnp.max(z, axis=1, keepdims=True)
        zs = z - m
        lse = jnp.log(jnp.sum(jnp.exp(zs), axis=1, keepdims=True))
        o_ref[...] = zs - lse


def _pool_head(h, batch2d, l0w, l0b, l1w, l1b):
    return pl.pallas_call(
        _pool_head_body,
        grid=(NP // G,),
        in_specs=[
            pl.BlockSpec((G, H), lambda i: (i, 0)),
            pl.BlockSpec((1, 1, G), lambda i: (i, 0, 0)),
            pl.BlockSpec((H, H), lambda i: (0, 0)),
            pl.BlockSpec((1, H), lambda i: (0, 0)),
            pl.BlockSpec((H, C), lambda i: (0, 0)),
            pl.BlockSpec((1, C), lambda i: (0, 0)),
        ],
        out_specs=pl.BlockSpec((G, C), lambda i: (0, 0)),
        out_shape=jax.ShapeDtypeStruct((G, C), jnp.float32),
        scratch_shapes=[
            pltpu.VMEM((G, H), jnp.float32),
            pltpu.VMEM((G, H), jnp.float32),
        ],
    )(h, batch2d, l0w, l0b, l1w, l1b)


def kernel(x, edge_index, edge_attr, batch,
           We0, be0, W0, b0,
           We1, be1, W1, b1,
           We2, be2, W2, b2,
           L0W, L0b, L1W, L1b):
    npad = E_PAD - E
    src = jnp.concatenate(
        [edge_index[0].astype(jnp.int32),
         jnp.arange(npad, dtype=jnp.int32) % N])
    dst = jnp.concatenate(
        [edge_index[1].astype(jnp.int32),
         N + jnp.arange(npad, dtype=jnp.int32) % (NP - N)])
    src = src.reshape(NW, NGRP, GCH, CHUNK)
    dst = dst.reshape(NW, NGRP, GCH, CHUNK)
    eap = jnp.concatenate(
        [edge_attr, jnp.zeros((npad, DE), jnp.float32)])

    xp = jnp.zeros((NP, D), jnp.float32).at[:N].set(x)
    zeros_np = jnp.zeros((NP, D), jnp.float32)

    wcat = jnp.concatenate([We0, We1, We2], axis=1)
    bcat = jnp.concatenate([be0, be1, be2]).reshape(1, 3 * D)
    eps = _eproj(eap, wcat, bcat)

    h = xp
    for ep, w, b in zip(eps, (W0, W1, W2), (b0, b1, b2)):
        aggs = _sc_layer(h, src, dst, ep, zeros_np)
        h = _update(h, aggs, w, b.reshape(1, H))

    batchp = jnp.full((NP,), G, jnp.int32).at[:N].set(batch.astype(jnp.int32))
    out = _pool_head(h, batchp.reshape(NP // G, 1, G),
                     L0W, L0b.reshape(1, H), L1W, L1b.reshape(1, C))
    return out


# speedup vs baseline: 4.4326x; 1.0629x over previous
import functools

import jax
import jax.numpy as jnp
from jax import lax
from jax.experimental import pallas as pl
from jax.experimental.pallas import tpu as pltpu
from jax.experimental.pallas import tpu_sc as plsc

N = 10000
E = 320000
D = 128
DE = 16
H = 128
C = 32
G = 128

NC = 2
NS = 16
NW = NC * NS
E_PER_W = E // NW
CHUNK = 40
NCHUNK = E_PER_W // CHUNK
NGRP = 5
GCH = NCHUNK // NGRP
NPAIR = GCH // 2
NA = 10240
ROWS_PER_SUB = NA // NS


def _sc_layer_body(h_hbm, src_hbm, dst_hbm, ep_hbm, zero_hbm, out_hbm,
                   src_v, dst_v, rows0, rows1, ep0, ep1, acc_sh,
                   g0, g1, e0, e1):
    cid = lax.axis_index("c")
    sid = lax.axis_index("s")
    wid = sid * NC + cid

    pltpu.sync_copy(zero_hbm.at[pl.ds(sid * ROWS_PER_SUB, ROWS_PER_SUB)],
                    acc_sh.at[pl.ds(sid * ROWS_PER_SUB, ROWS_PER_SUB)])
    plsc.subcore_barrier()

    ep_base = wid * E_PER_W

    def compute(rows_v, ep_v):
        @plsc.parallel_loop(0, CHUNK, unroll=4)
        def row_body(r):
            for j in range(D // 16):
                s = pl.ds(j * 16, 16)
                v = rows_v[r, s] + ep_v[r, s]
                rows_v[r, s] = jnp.maximum(v, 0.0)

    for grp in range(NGRP):
        goff = grp * GCH

        def issue(c_grp, rows_v, ep_v, gsem, esem, goff=goff):
            pltpu.async_copy(h_hbm.at[src_v.at[c_grp]], rows_v, gsem)
            pltpu.async_copy(
                ep_hbm.at[pl.ds(ep_base + (goff + c_grp) * CHUNK, CHUNK)],
                ep_v, esem)

        pltpu.sync_copy(src_hbm.at[wid, grp], src_v)
        pltpu.sync_copy(dst_hbm.at[wid, grp], dst_v)
        issue(0, rows0, ep0, g0, e0)

        def pair_body(k, carry):
            c_even = 2 * k
            c_odd = 2 * k + 1
            issue(c_odd, rows1, ep1, g1, e1)
            pltpu.make_async_copy(h_hbm.at[src_v.at[c_even]], rows0, g0).wait()
            pltpu.make_async_copy(
                ep_hbm.at[pl.ds(ep_base, CHUNK)], ep0, e0).wait()
            compute(rows0, ep0)
            pltpu.sync_copy(rows0, acc_sh.at[dst_v.at[c_even]], add=True)

            @pl.when(k < NPAIR - 1)
            def _():
                issue(c_even + 2, rows0, ep0, g0, e0)

            pltpu.make_async_copy(h_hbm.at[src_v.at[c_odd]], rows1, g1).wait()
            pltpu.make_async_copy(
                ep_hbm.at[pl.ds(ep_base, CHUNK)], ep1, e1).wait()
            compute(rows1, ep1)
            pltpu.sync_copy(rows1, acc_sh.at[dst_v.at[c_odd]], add=True)
            return carry

        lax.fori_loop(0, NPAIR, pair_body, 0)

    plsc.subcore_barrier()
    pltpu.sync_copy(acc_sh.at[pl.ds(sid * ROWS_PER_SUB, ROWS_PER_SUB)],
                    out_hbm.at[cid, pl.ds(sid * ROWS_PER_SUB, ROWS_PER_SUB)])


_sc_layer = pl.kernel(
    _sc_layer_body,
    out_type=jax.ShapeDtypeStruct((NC, NA, D), jnp.float32),
    mesh=plsc.VectorSubcoreMesh(core_axis_name="c", subcore_axis_name="s"),
    scratch_types=[
        pltpu.VMEM((GCH, CHUNK), jnp.int32),
        pltpu.VMEM((GCH, CHUNK), jnp.int32),
        pltpu.VMEM((CHUNK, D), jnp.float32),
        pltpu.VMEM((CHUNK, D), jnp.float32),
        pltpu.VMEM((CHUNK, D), jnp.float32),
        pltpu.VMEM((CHUNK, D), jnp.float32),
        pltpu.VMEM_SHARED((NA, D), jnp.float32),
        pltpu.SemaphoreType.DMA,
        pltpu.SemaphoreType.DMA,
        pltpu.SemaphoreType.DMA,
        pltpu.SemaphoreType.DMA,
    ],
)


def _eproj_body(ea_ref, w_ref, b_ref, o0_ref, o1_ref, o2_ref):
    prod = jnp.dot(ea_ref[...], w_ref[...],
                   preferred_element_type=jnp.float32) + b_ref[...]
    o0_ref[...] = prod[:, 0:D]
    o1_ref[...] = prod[:, D:2 * D]
    o2_ref[...] = prod[:, 2 * D:3 * D]


_EBLK = 2000


def _eproj(edge_attr, wcat, bcat):
    return pl.pallas_call(
        _eproj_body,
        grid=(E // _EBLK,),
        in_specs=[
            pl.BlockSpec((_EBLK, DE), lambda i: (i, 0)),
            pl.BlockSpec((DE, 3 * D), lambda i: (0, 0)),
            pl.BlockSpec((1, 3 * D), lambda i: (0, 0)),
        ],
        out_specs=[
            pl.BlockSpec((_EBLK, D), lambda i: (i, 0)),
            pl.BlockSpec((_EBLK, D), lambda i: (i, 0)),
            pl.BlockSpec((_EBLK, D), lambda i: (i, 0)),
        ],
        out_shape=[jax.ShapeDtypeStruct((E, D), jnp.float32)] * 3,
    )(edge_attr, wcat, bcat)


def _update_body(h_ref, agg_ref, w_ref, b_ref, o_ref):
    s = h_ref[...] + agg_ref[0] + agg_ref[1]
    m = jnp.dot(s, w_ref[...], preferred_element_type=jnp.float32) + b_ref[...]
    o_ref[...] = jnp.where(m > 0, m, 0.01 * m)


_NBLK = 1000


def _update(h, aggs, w, b):
    return pl.pallas_call(
        _update_body,
        grid=(N // _NBLK,),
        in_specs=[
            pl.BlockSpec((_NBLK, D), lambda i: (i, 0)),
            pl.BlockSpec((NC, _NBLK, D), lambda i: (0, i, 0)),
            pl.BlockSpec((D, H), lambda i: (0, 0)),
            pl.BlockSpec((1, H), lambda i: (0, 0)),
        ],
        out_specs=pl.BlockSpec((_NBLK, H), lambda i: (i, 0)),
        out_shape=jax.ShapeDtypeStruct((N, H), jnp.float32),
    )(h, aggs, w, b)


def _pool_head_body(h_ref, batch_ref, l0w_ref, l0b_ref, l1w_ref, l1b_ref,
                    o_ref, sums_ref, cnts_ref):
    i = pl.program_id(0)

    @pl.when(i == 0)
    def _():
        sums_ref[...] = jnp.zeros_like(sums_ref)
        cnts_ref[...] = jnp.zeros_like(cnts_ref)

    gids = lax.broadcasted_iota(jnp.int32, (G, _PBLK), 0)
    mask = (gids == batch_ref[0]).astype(jnp.float32)
    sums_ref[...] += jnp.dot(mask, h_ref[...],
                             preferred_element_type=jnp.float32)
    cnts_ref[...] += jnp.dot(mask, jnp.ones((_PBLK, H), jnp.float32),
                             preferred_element_type=jnp.float32)

    @pl.when(i == pl.num_programs(0) - 1)
    def _():
        pooled = sums_ref[...] / jnp.maximum(cnts_ref[...], 1.0)
        z = jnp.dot(pooled, l0w_ref[...],
                    preferred_element_type=jnp.float32) + l0b_ref[...]
        z = jnp.where(z > 0, z, 0.01 * z)
        z = jnp.dot(z, l1w_ref[...],
                    preferred_element_type=jnp.float32) + l1b_ref[...]
        m = jnp.max(z, axis=1, keepdims=True)
        zs = z - m
        lse = jnp.log(jnp.sum(jnp.exp(zs), axis=1, keepdims=True))
        o_ref[...] = zs - lse


_PBLK = 1000


def _pool_head(h, batch2d, l0w, l0b, l1w, l1b):
    return pl.pallas_call(
        _pool_head_body,
        grid=(N // _PBLK,),
        in_specs=[
            pl.BlockSpec((_PBLK, H), lambda i: (i, 0)),
            pl.BlockSpec((1, 1, _PBLK), lambda i: (i, 0, 0)),
            pl.BlockSpec((H, H), lambda i: (0, 0)),
            pl.BlockSpec((1, H), lambda i: (0, 0)),
            pl.BlockSpec((H, C), lambda i: (0, 0)),
            pl.BlockSpec((1, C), lambda i: (0, 0)),
        ],
        out_specs=pl.BlockSpec((G, C), lambda i: (0, 0)),
        out_shape=jax.ShapeDtypeStruct((G, C), jnp.float32),
        scratch_shapes=[
            pltpu.VMEM((G, H), jnp.float32),
            pltpu.VMEM((G, H), jnp.float32),
        ],
    )(h, batch2d, l0w, l0b, l1w, l1b)


def kernel(x, edge_index, edge_attr, batch,
           We0, be0, W0, b0,
           We1, be1, W1, b1,
           We2, be2, W2, b2,
           L0W, L0b, L1W, L1b):
    src = edge_index[0].astype(jnp.int32).reshape(NW, NGRP, GCH, CHUNK)
    dst = edge_index[1].astype(jnp.int32).reshape(NW, NGRP, GCH, CHUNK)
    zeros_n = jnp.zeros((NA, D), jnp.float32)

    wcat = jnp.concatenate([We0, We1, We2], axis=1)
    bcat = jnp.concatenate([be0, be1, be2]).reshape(1, 3 * D)
    eps = _eproj(edge_attr, wcat, bcat)

    h = x
    for ep, w, b in zip(eps, (W0, W1, W2), (b0, b1, b2)):
        aggs = _sc_layer(h, src, dst, ep, zeros_n)
        h = _update(h, aggs, w, b.reshape(1, H))

    out = _pool_head(h, batch.astype(jnp.int32).reshape(N // _PBLK, 1, _PBLK),
                     L0W, L0b.reshape(1, H), L1W, L1b.reshape(1, C))
    return out


# speedup vs baseline: 4.4332x; 1.0001x over previous
import functools

import jax
import jax.numpy as jnp
from jax import lax
from jax.experimental import pallas as pl
from jax.experimental.pallas import tpu as pltpu
from jax.experimental.pallas import tpu_sc as plsc

N = 10000
E = 320000
D = 128
DE = 16
H = 128
C = 32
G = 128

NC = 2
NS = 16
NW = NC * NS
E_PER_W = E // NW
CHUNK = 40
NCHUNK = E_PER_W // CHUNK
NGRP = 5
GCH = NCHUNK // NGRP
NPAIR = GCH // 2
NA = 10240
ROWS_PER_SUB = NA // NS


def _sc_layer_body(h_hbm, src_hbm, dst_hbm, ep_hbm, zero_hbm, out_hbm,
                   src_v, dst_v, rows0, rows1, ep0, ep1, acc_sh,
                   g0, g1, e0, e1):
    cid = lax.axis_index("c")
    sid = lax.axis_index("s")
    wid = sid * NC + cid

    pltpu.sync_copy(zero_hbm.at[pl.ds(sid * ROWS_PER_SUB, ROWS_PER_SUB)],
                    acc_sh.at[pl.ds(sid * ROWS_PER_SUB, ROWS_PER_SUB)])
    plsc.subcore_barrier()

    ep_base = wid * E_PER_W

    def compute(rows_v, ep_v):
        @plsc.parallel_loop(0, CHUNK, unroll=4)
        def row_body(r):
            for j in range(D // 16):
                s = pl.ds(j * 16, 16)
                v = rows_v[r, s] + ep_v[r, s]
                rows_v[r, s] = jnp.maximum(v, 0.0)

    for grp in range(NGRP):
        goff = grp * GCH

        def issue(c_grp, rows_v, ep_v, gsem, esem, goff=goff):
            pltpu.async_copy(h_hbm.at[src_v.at[c_grp]], rows_v, gsem)
            pltpu.async_copy(
                ep_hbm.at[pl.ds(ep_base + (goff + c_grp) * CHUNK, CHUNK)],
                ep_v, esem)

        pltpu.sync_copy(src_hbm.at[wid, grp], src_v)
        pltpu.sync_copy(dst_hbm.at[wid, grp], dst_v)
        issue(0, rows0, ep0, g0, e0)

        def pair_body(k, carry):
            c_even = 2 * k
            c_odd = 2 * k + 1
            issue(c_odd, rows1, ep1, g1, e1)
            pltpu.make_async_copy(h_hbm.at[src_v.at[c_even]], rows0, g0).wait()
            pltpu.make_async_copy(
                ep_hbm.at[pl.ds(ep_base, CHUNK)], ep0, e0).wait()
            compute(rows0, ep0)
            pltpu.sync_copy(rows0, acc_sh.at[dst_v.at[c_even]], add=True)

            @pl.when(k < NPAIR - 1)
            def _():
                issue(c_even + 2, rows0, ep0, g0, e0)

            pltpu.make_async_copy(h_hbm.at[src_v.at[c_odd]], rows1, g1).wait()
            pltpu.make_async_copy(
                ep_hbm.at[pl.ds(ep_base, CHUNK)], ep1, e1).wait()
            compute(rows1, ep1)
            pltpu.sync_copy(rows1, acc_sh.at[dst_v.at[c_odd]], add=True)
            return carry

        lax.fori_loop(0, NPAIR, pair_body, 0)

    plsc.subcore_barrier()
    pltpu.sync_copy(acc_sh.at[pl.ds(sid * ROWS_PER_SUB, ROWS_PER_SUB)],
                    out_hbm.at[cid, pl.ds(sid * ROWS_PER_SUB, ROWS_PER_SUB)])


_sc_layer = pl.kernel(
    _sc_layer_body,
    out_type=jax.ShapeDtypeStruct((NC, NA, D), jnp.float32),
    mesh=plsc.VectorSubcoreMesh(core_axis_name="c", subcore_axis_name="s"),
    scratch_types=[
        pltpu.VMEM((GCH, CHUNK), jnp.int32),
        pltpu.VMEM((GCH, CHUNK), jnp.int32),
        pltpu.VMEM((CHUNK, D), jnp.float32),
        pltpu.VMEM((CHUNK, D), jnp.float32),
        pltpu.VMEM((CHUNK, D), jnp.float32),
        pltpu.VMEM((CHUNK, D), jnp.float32),
        pltpu.VMEM_SHARED((NA, D), jnp.float32),
        pltpu.SemaphoreType.DMA,
        pltpu.SemaphoreType.DMA,
        pltpu.SemaphoreType.DMA,
        pltpu.SemaphoreType.DMA,
    ],
)


def _eproj_body(ea_ref, w_ref, b_ref, o_ref):
    o_ref[...] = jnp.dot(ea_ref[...], w_ref[...],
                         preferred_element_type=jnp.float32) + b_ref[...]


_EBLK = 2000


def _eproj(edge_attr, w, b):
    return pl.pallas_call(
        _eproj_body,
        grid=(E // _EBLK,),
        in_specs=[
            pl.BlockSpec((_EBLK, DE), lambda i: (i, 0)),
            pl.BlockSpec((DE, D), lambda i: (0, 0)),
            pl.BlockSpec((1, D), lambda i: (0, 0)),
        ],
        out_specs=pl.BlockSpec((_EBLK, D), lambda i: (i, 0)),
        out_shape=jax.ShapeDtypeStruct((E, D), jnp.float32),
    )(edge_attr, w, b)


def _update_body(h_ref, agg_ref, w_ref, b_ref, o_ref):
    s = h_ref[...] + agg_ref[0] + agg_ref[1]
    m = jnp.dot(s, w_ref[...], preferred_element_type=jnp.float32) + b_ref[...]
    o_ref[...] = jnp.where(m > 0, m, 0.01 * m)


_NBLK = 1000


def _update(h, aggs, w, b):
    return pl.pallas_call(
        _update_body,
        grid=(N // _NBLK,),
        in_specs=[
            pl.BlockSpec((_NBLK, D), lambda i: (i, 0)),
            pl.BlockSpec((NC, _NBLK, D), lambda i: (0, i, 0)),
            pl.BlockSpec((D, H), lambda i: (0, 0)),
            pl.BlockSpec((1, H), lambda i: (0, 0)),
        ],
        out_specs=pl.BlockSpec((_NBLK, H), lambda i: (i, 0)),
        out_shape=jax.ShapeDtypeStruct((N, H), jnp.float32),
    )(h, aggs, w, b)


def _pool_head_body(h_ref, batch_ref, l0w_ref, l0b_ref, l1w_ref, l1b_ref,
                    o_ref, sums_ref, cnts_ref):
    i = pl.program_id(0)

    @pl.when(i == 0)
    def _():
        sums_ref[...] = jnp.zeros_like(sums_ref)
        cnts_ref[...] = jnp.zeros_like(cnts_ref)

    gids = lax.broadcasted_iota(jnp.int32, (G, _PBLK), 0)
    mask = (gids == batch_ref[0]).astype(jnp.float32)
    sums_ref[...] += jnp.dot(mask, h_ref[...],
                             preferred_element_type=jnp.float32)
    cnts_ref[...] += jnp.dot(mask, jnp.ones((_PBLK, H), jnp.float32),
                             preferred_element_type=jnp.float32)

    @pl.when(i == pl.num_programs(0) - 1)
    def _():
        pooled = sums_ref[...] / jnp.maximum(cnts_ref[...], 1.0)
        z = jnp.dot(pooled, l0w_ref[...],
                    preferred_element_type=jnp.float32) + l0b_ref[...]
        z = jnp.where(z > 0, z, 0.01 * z)
        z = jnp.dot(z, l1w_ref[...],
                    preferred_element_type=jnp.float32) + l1b_ref[...]
        m = jnp.max(z, axis=1, keepdims=True)
        zs = z - m
        lse = jnp.log(jnp.sum(jnp.exp(zs), axis=1, keepdims=True))
        o_ref[...] = zs - lse


_PBLK = 1000


def _pool_head(h, batch2d, l0w, l0b, l1w, l1b):
    return pl.pallas_call(
        _pool_head_body,
        grid=(N // _PBLK,),
        in_specs=[
            pl.BlockSpec((_PBLK, H), lambda i: (i, 0)),
            pl.BlockSpec((1, 1, _PBLK), lambda i: (i, 0, 0)),
            pl.BlockSpec((H, H), lambda i: (0, 0)),
            pl.BlockSpec((1, H), lambda i: (0, 0)),
            pl.BlockSpec((H, C), lambda i: (0, 0)),
            pl.BlockSpec((1, C), lambda i: (0, 0)),
        ],
        out_specs=pl.BlockSpec((G, C), lambda i: (0, 0)),
        out_shape=jax.ShapeDtypeStruct((G, C), jnp.float32),
        scratch_shapes=[
            pltpu.VMEM((G, H), jnp.float32),
            pltpu.VMEM((G, H), jnp.float32),
        ],
    )(h, batch2d, l0w, l0b, l1w, l1b)


def kernel(x, edge_index, edge_attr, batch,
           We0, be0, W0, b0,
           We1, be1, W1, b1,
           We2, be2, W2, b2,
           L0W, L0b, L1W, L1b):
    src = edge_index[0].astype(jnp.int32).reshape(NW, NGRP, GCH, CHUNK)
    dst = edge_index[1].astype(jnp.int32).reshape(NW, NGRP, GCH, CHUNK)
    zeros_n = jnp.zeros((NA, D), jnp.float32)

    h = x
    for we, be, w, b in ((We0, be0, W0, b0), (We1, be1, W1, b1),
                         (We2, be2, W2, b2)):
        ep = _eproj(edge_attr, we, be.reshape(1, D))
        aggs = _sc_layer(h, src, dst, ep, zeros_n)
        h = _update(h, aggs, w, b.reshape(1, H))

    out = _pool_head(h, batch.astype(jnp.int32).reshape(N // _PBLK, 1, _PBLK),
                     L0W, L0b.reshape(1, H), L1W, L1b.reshape(1, C))
    return out
